# baseline (speedup 1.0000x reference)
import functools

import numpy as np
import jax
import jax.numpy as jnp
from jax.experimental import pallas as pl
from jax.experimental.pallas import tpu as pltpu

_KSIZE = 4
_STRIDE = 2
_PAD = 1
_CHANNELS = (6, 16, 32, 32)


def _out_hw(size):
    return (size + 2 * _PAD - _KSIZE) // _STRIDE + 1


@functools.lru_cache(maxsize=None)
def _wfold_structure(w_in):
    w_out = _out_hw(w_in)
    t = np.zeros((w_in, _KSIZE, w_out), np.float32)
    for kw in range(_KSIZE):
        for ow in range(w_out):
            iw = ow * _STRIDE - _PAD + kw
            if 0 <= iw < w_in:
                t[iw, kw, ow] = 1.0
    return t


@functools.lru_cache(maxsize=None)
def _hmajor_to_nchw_perm(ho, cout, wo):
    p = np.zeros((ho * cout * wo, cout * ho * wo), np.float32)
    for oh in range(ho):
        for c in range(cout):
            for ow in range(wo):
                p[(oh * cout + c) * wo + ow, (c * ho + oh) * wo + ow] = 1.0
    return p


def _row_window(oh, h_in):
    lo = max(_STRIDE * oh - _PAD, 0)
    hi = min(_STRIDE * oh - _PAD + _KSIZE, h_in)
    return lo, hi, lo - (_STRIDE * oh - _PAD)


def _conv_layer(x_pieces, a_ref, b_ref, h_in, row_lanes):
    h_out = _out_hw(h_in)
    out = []
    for oh in range(h_out):
        lo, hi, k0 = _row_window(oh, h_in)
        if isinstance(x_pieces, list):
            xs = jnp.concatenate(x_pieces[lo:hi], axis=1)
        else:
            xs = x_pieces[:, lo * row_lanes:hi * row_lanes]
        a_sl = a_ref[k0 * row_lanes:(k0 + (hi - lo)) * row_lanes, :]
        y = jnp.dot(xs, a_sl, preferred_element_type=jnp.float32)
        out.append(jnp.maximum(y + b_ref[...], 0.0))
    return out


def _conv_layer1(x_ref, a_ref, b_ref, cin, h_in, w_in):
    h_out = _out_hw(h_in)
    hw = h_in * w_in
    xv = x_ref[...].astype(jnp.bfloat16)
    out = []
    for oh in range(h_out):
        lo, hi, k0 = _row_window(oh, h_in)
        acc = None
        for ci in range(cin):
            xs = xv[:, ci * hw + lo * w_in:ci * hw + hi * w_in]
            a_sl = a_ref[ci * _KSIZE * w_in + k0 * w_in:
                         ci * _KSIZE * w_in + (k0 + hi - lo) * w_in, :]
            p = jnp.dot(xs, a_sl, preferred_element_type=jnp.float32)
            acc = p if acc is None else acc + p
        out.append(jnp.maximum(acc + b_ref[...], 0.0))
    return out


def _cnn_kernel(x_ref, a1_ref, b1_ref, a2_ref, b2_ref, a3_ref, b3_ref,
                p1_ref, p2_ref, p3_ref,
                o1_ref, o2_ref, o3_ref):
    ys1 = _conv_layer1(x_ref, a1_ref, b1_ref, cin=6, h_in=20, w_in=20)
    y1b = [y.astype(jnp.bfloat16) for y in ys1]
    o1_ref[...] = jnp.dot(jnp.concatenate(y1b, axis=1), p1_ref[...],
                          preferred_element_type=jnp.float32).astype(jnp.bfloat16)

    ys2 = _conv_layer(y1b, a2_ref, b2_ref, h_in=10, row_lanes=160)
    y2b = [y.astype(jnp.bfloat16) for y in ys2]
    o2_ref[...] = jnp.dot(jnp.concatenate(y2b, axis=1), p2_ref[...],
                          preferred_element_type=jnp.float32).astype(jnp.bfloat16)

    ys3 = _conv_layer(y2b, a3_ref, b3_ref, h_in=5, row_lanes=160)
    y3b = [y.astype(jnp.bfloat16) for y in ys3]
    o3_ref[...] = jnp.dot(jnp.concatenate(y3b, axis=1), p3_ref[...],
                          preferred_element_type=jnp.float32)


def kernel(x, conv1_w, conv1_b, conv2_w, conv2_b, conv3_w, conv3_b):
    n, c_in, h, w = x.shape
    assert c_in == _CHANNELS[0]

    dims = []
    hh, ww = h, w
    for li in range(3):
        ho, wo = _out_hw(hh), _out_hw(ww)
        dims.append((_CHANNELS[li], _CHANNELS[li + 1], hh, ww, ho, wo))
        hh, ww = ho, wo

    def fold(li, wgt, bias, order):
        cin, cout, hi, wi, ho, wo = dims[li]
        t = jnp.asarray(_wfold_structure(wi))
        a = jnp.einsum(f'oikl,wlv->{order}ov', wgt, t)
        a = a.reshape(-1, cout * wo).astype(jnp.bfloat16)
        brow = jnp.broadcast_to(bias[:, None], (cout, wo))
        return a, brow.reshape(1, cout * wo).astype(jnp.float32)

    a1, b1 = fold(0, conv1_w, conv1_b, 'ikw')
    a2, b2 = fold(1, conv2_w, conv2_b, 'kiw')
    a3, b3 = fold(2, conv3_w, conv3_b, 'kiw')

    xh = x.reshape(n, c_in * h * w)

    p1 = jnp.asarray(_hmajor_to_nchw_perm(10, 16, 10), jnp.bfloat16)
    p2 = jnp.asarray(_hmajor_to_nchw_perm(5, 32, 5), jnp.bfloat16)
    p3 = jnp.asarray(_hmajor_to_nchw_perm(2, 32, 2), jnp.bfloat16)

    bn = 128 if n % 128 == 0 else n
    steps = n // bn
    resident = lambda arr: pl.BlockSpec(arr.shape, lambda b: (0, 0))

    osizes = [dims[li][1] * dims[li][4] * dims[li][5] for li in range(3)]
    odtypes = [jnp.bfloat16, jnp.bfloat16, jnp.float32]
    o1, o2, o3 = pl.pallas_call(
        _cnn_kernel,
        grid=(steps,),
        out_shape=tuple(
            jax.ShapeDtypeStruct((n, fs), dt) for fs, dt in zip(osizes, odtypes)),
        in_specs=[
            pl.BlockSpec((bn, c_in * h * w), lambda b: (b, 0)),
            resident(a1), resident(b1),
            resident(a2), resident(b2),
            resident(a3), resident(b3),
            resident(p1), resident(p2), resident(p3),
        ],
        out_specs=tuple(
            pl.BlockSpec((bn, fs), lambda b: (b, 0)) for fs in osizes),
        compiler_params=pltpu.CompilerParams(
            dimension_semantics=("arbitrary",)),
    )(xh, a1, b1, a2, b2, a3, b3, p1, p2, p3)

    feat1 = o1.reshape(n, dims[0][1], dims[0][4], dims[0][5]).astype(jnp.float32)
    feat2 = o2.reshape(n, dims[1][1], dims[1][4], dims[1][5]).astype(jnp.float32)
    feat3 = o3.reshape(n, dims[2][1], dims[2][4], dims[2][5])
    flat = o3
    return flat, [feat1, feat2, feat3]


# speedup vs baseline: 1.1304x; 1.1304x over previous
import functools

import numpy as np
import jax
import jax.numpy as jnp
from jax.experimental import pallas as pl
from jax.experimental.pallas import tpu as pltpu

_KSIZE = 4
_STRIDE = 2
_PAD = 1
_CHANNELS = (6, 16, 32, 32)


def _out_hw(size):
    return (size + 2 * _PAD - _KSIZE) // _STRIDE + 1


@functools.lru_cache(maxsize=None)
def _wfold_structure(w_in):
    w_out = _out_hw(w_in)
    t = np.zeros((w_in, _KSIZE, w_out), np.float32)
    for kw in range(_KSIZE):
        for ow in range(w_out):
            iw = ow * _STRIDE - _PAD + kw
            if 0 <= iw < w_in:
                t[iw, kw, ow] = 1.0
    return t


def _row_window(oh, h_in):
    lo = max(_STRIDE * oh - _PAD, 0)
    hi = min(_STRIDE * oh - _PAD + _KSIZE, h_in)
    return lo, hi, lo - (_STRIDE * oh - _PAD)


def _conv_layer(x_pieces, a_ref, b_ref, h_in, row_lanes):
    h_out = _out_hw(h_in)
    out = []
    for oh in range(h_out):
        lo, hi, k0 = _row_window(oh, h_in)
        xs = jnp.concatenate(x_pieces[lo:hi], axis=1)
        a_sl = a_ref[k0 * row_lanes:(k0 + (hi - lo)) * row_lanes, :]
        y = jnp.dot(xs, a_sl, preferred_element_type=jnp.float32)
        out.append(jnp.maximum(y + b_ref[...], 0.0))
    return out


def _conv_layer1(x_ref, a_ref, b_ref, cin, h_in, w_in):
    h_out = _out_hw(h_in)
    hw = h_in * w_in
    xv = x_ref[...].astype(jnp.bfloat16)
    out = []
    for oh in range(h_out):
        lo, hi, k0 = _row_window(oh, h_in)
        acc = None
        for ci in range(cin):
            xs = xv[:, ci * hw + lo * w_in:ci * hw + hi * w_in]
            a_sl = a_ref[ci * _KSIZE * w_in + k0 * w_in:
                         ci * _KSIZE * w_in + (k0 + hi - lo) * w_in, :]
            p = jnp.dot(xs, a_sl, preferred_element_type=jnp.float32)
            acc = p if acc is None else acc + p
        out.append(jnp.maximum(acc + b_ref[...], 0.0))
    return out


def _cnn_kernel(x_ref, a1_ref, b1_ref, a2_ref, b2_ref, a3_ref, b3_ref,
                o1_ref, o2_ref, o3_ref):
    ys1 = _conv_layer1(x_ref, a1_ref, b1_ref, cin=6, h_in=20, w_in=20)
    y1b = [y.astype(jnp.bfloat16) for y in ys1]
    o1_ref[...] = jnp.concatenate(y1b, axis=1)

    ys2 = _conv_layer(y1b, a2_ref, b2_ref, h_in=10, row_lanes=160)
    y2b = [y.astype(jnp.bfloat16) for y in ys2]
    o2_ref[...] = jnp.concatenate(y2b, axis=1)

    ys3 = _conv_layer(y2b, a3_ref, b3_ref, h_in=5, row_lanes=160)
    y3b = [y.astype(jnp.bfloat16) for y in ys3]
    o3_ref[...] = jnp.concatenate(y3b, axis=1)


def kernel(x, conv1_w, conv1_b, conv2_w, conv2_b, conv3_w, conv3_b):
    n, c_in, h, w = x.shape
    assert c_in == _CHANNELS[0]

    dims = []
    hh, ww = h, w
    for li in range(3):
        ho, wo = _out_hw(hh), _out_hw(ww)
        dims.append((_CHANNELS[li], _CHANNELS[li + 1], hh, ww, ho, wo))
        hh, ww = ho, wo

    def fold(li, wgt, bias, order):
        cin, cout, hi, wi, ho, wo = dims[li]
        t = jnp.asarray(_wfold_structure(wi))
        a = jnp.einsum(f'oikl,wlv->{order}ov', wgt, t)
        a = a.reshape(-1, cout * wo).astype(jnp.bfloat16)
        brow = jnp.broadcast_to(bias[:, None], (cout, wo))
        return a, brow.reshape(1, cout * wo).astype(jnp.float32)

    a1, b1 = fold(0, conv1_w, conv1_b, 'ikw')
    a2, b2 = fold(1, conv2_w, conv2_b, 'kiw')
    a3, b3 = fold(2, conv3_w, conv3_b, 'kiw')

    xh = x.reshape(n, c_in * h * w)

    bn = 128 if n % 128 == 0 else n
    steps = n // bn
    resident = lambda arr: pl.BlockSpec(arr.shape, lambda b: (0, 0))

    osizes = [dims[li][1] * dims[li][4] * dims[li][5] for li in range(3)]
    o1, o2, o3 = pl.pallas_call(
        _cnn_kernel,
        grid=(steps,),
        out_shape=tuple(
            jax.ShapeDtypeStruct((n, fs), jnp.bfloat16) for fs in osizes),
        in_specs=[
            pl.BlockSpec((bn, c_in * h * w), lambda b: (b, 0)),
            resident(a1), resident(b1),
            resident(a2), resident(b2),
            resident(a3), resident(b3),
        ],
        out_specs=tuple(
            pl.BlockSpec((bn, fs), lambda b: (b, 0)) for fs in osizes),
        compiler_params=pltpu.CompilerParams(
            dimension_semantics=("parallel",)),
    )(xh, a1, b1, a2, b2, a3, b3)

    def to_nchw(o, li):
        cin, cout, hi, wi, ho, wo = dims[li]
        o = o.reshape(n, ho, cout, wo).transpose(0, 2, 1, 3)
        return o.astype(jnp.float32)

    feat1 = to_nchw(o1, 0)
    feat2 = to_nchw(o2, 1)
    feat3 = to_nchw(o3, 2)
    flat = feat3.reshape(n, osizes[2])
    return flat, [feat1, feat2, feat3]
